# Initial kernel scaffold; baseline (speedup 1.0000x reference)
import jax
import jax.numpy as jnp
from jax.experimental import pallas as pl


def kernel(h, e, snorm_n, snorm_e, edge_index, params):
    raise NotImplementedError("write your pallas kernel here")


# speedup vs baseline: 1.6658x; 1.6658x over previous
import functools

import jax
import jax.numpy as jnp
from jax import lax
from jax.experimental import pallas as pl
from jax.experimental.pallas import tpu as pltpu
from jax.experimental.pallas import tpu_sc as plsc

_N = 10000
_E = 320000
_IN_DIM = 128
_HID = 256
_L = 4


_CAP = 32768
_RT = 312
_RT_LAST = _N - 31 * _RT
_RPAD = 336
_DUMMY = 328
_IBLK = 2000
_NBLK = _E // _IBLK
_SB = 2048
_CK = 128
_CP = pltpu.CompilerParams(needs_layout_passes=False)


def _make_preproc():
    mesh = plsc.VectorSubcoreMesh(core_axis_name="c", subcore_axis_name="s")

    @functools.partial(
        pl.kernel, mesh=mesh,
        out_type=[jax.ShapeDtypeStruct((32, _CAP), jnp.int32),
                  jax.ShapeDtypeStruct((32, _CAP), jnp.int32),
                  jax.ShapeDtypeStruct((32, 16), jnp.int32)],
        compiler_params=_CP,
        scratch_types=[
            pltpu.VMEM((_IBLK,), jnp.int32),
            pltpu.VMEM((_IBLK,), jnp.int32),
            pltpu.VMEM((_CAP,), jnp.int32),
            pltpu.VMEM((_CAP,), jnp.int32),
            pltpu.VMEM((16,), jnp.int32),
        ],
    )
    def prep(src_hbm, dst_hbm, csrc_hbm, cldst_hbm, cnt_hbm,
             sblk, dblk, cbs, cbl, cnt_v):
        c = lax.axis_index("c")
        s = lax.axis_index("s")
        wid = s * 2 + c
        lo = wid * _RT
        hi = jnp.where(wid == 31, _N, lo + _RT)

        def _init(i, carry):
            cbs[pl.ds(i * 16, 16)] = jnp.zeros((16,), jnp.int32)
            cbl[pl.ds(i * 16, 16)] = jnp.full((16,), _DUMMY, jnp.int32)
            return carry
        lax.fori_loop(0, _CAP // 16, _init, 0)

        def _blk(b, ptr):
            pltpu.sync_copy(src_hbm.at[pl.ds(b * _IBLK, _IBLK)], sblk)
            pltpu.sync_copy(dst_hbm.at[pl.ds(b * _IBLK, _IBLK)], dblk)

            def _grp(g, ptr2):
                sv = sblk[pl.ds(g * 16, 16)]
                d = dblk[pl.ds(g * 16, 16)]
                m = (d >= lo) & (d < hi)
                plsc.store_compressed(cbs.at[pl.ds(ptr2, 16)], sv, mask=m)
                plsc.store_compressed(cbl.at[pl.ds(ptr2, 16)], d - lo, mask=m)
                pc = plsc.all_reduce_population_count(m)
                pc_s = pc if pc.ndim == 0 else pc[0]
                return ptr2 + pc_s
            return lax.fori_loop(0, _IBLK // 16, _grp, ptr)

        ptr = lax.fori_loop(0, _NBLK, _blk, 0)
        cnt_v[...] = jnp.zeros((16,), jnp.int32) + ptr
        pltpu.sync_copy(cbs, csrc_hbm.at[wid])
        pltpu.sync_copy(cbl, cldst_hbm.at[wid])
        pltpu.sync_copy(cnt_v, cnt_hbm.at[wid])

    return prep


def _make_seg_sum(depth):
    mesh = plsc.VectorSubcoreMesh(core_axis_name="c", subcore_axis_name="s")

    @functools.partial(
        pl.kernel, mesh=mesh,
        out_type=jax.ShapeDtypeStruct((_N, depth), jnp.float32),
        compiler_params=_CP,
        scratch_types=[
            pltpu.VMEM((_SB,), jnp.int32),
            pltpu.VMEM((_SB,), jnp.int32),
            pltpu.VMEM((_CK, depth), jnp.float32),
            pltpu.VMEM((_RPAD, depth), jnp.float32),
            pltpu.VMEM((16,), jnp.int32),
            pltpu.SemaphoreType.DMA,
        ],
    )
    def seg(h_hbm, csrc_hbm, cldst_hbm, cnt_hbm, out_hbm,
            sblk, lblk, rows_v, acc, cnt_v, sem):
        c = lax.axis_index("c")
        s = lax.axis_index("s")
        wid = s * 2 + c
        base_row = wid * _RT

        def _zero(i, carry):
            for j in range(depth // 16):
                acc[i, pl.ds(j * 16, 16)] = jnp.zeros((16,), jnp.float32)
            return carry
        lax.fori_loop(0, _RPAD, _zero, 0)

        pltpu.sync_copy(cnt_hbm.at[wid], cnt_v)
        cnt = jnp.max(cnt_v[...])
        nblk = (cnt + _SB - 1) // _SB
        cols = lax.iota(jnp.int32, 16)

        def _blk(b, carry):
            pltpu.sync_copy(csrc_hbm.at[wid, pl.ds(b * _SB, _SB)], sblk)
            pltpu.sync_copy(cldst_hbm.at[wid, pl.ds(b * _SB, _SB)], lblk)
            rem = cnt - b * _SB
            nck = jnp.clip((rem + _CK - 1) // _CK, 0, _SB // _CK)

            def _chunk(k, carry2):
                pltpu.async_copy(
                    h_hbm.at[sblk.at[pl.ds(k * _CK, _CK)]], rows_v,
                    sem).wait()
                for g in range(_CK // 16):
                    ld16 = lblk[pl.ds(k * _CK + g * 16, 16)]
                    ok = (ld16 >= 0) & (ld16 < _DUMMY)
                    ld16 = jnp.where(ok, ld16, _DUMMY)
                    for el in range(16):
                        rsc = jnp.sum(jnp.where(cols == el, ld16, 0))
                        r16 = jnp.zeros((16,), jnp.int32) + rsc
                        e = g * 16 + el
                        for j in range(depth // 16):
                            v = rows_v[e, pl.ds(j * 16, 16)]
                            plsc.addupdate_scatter(
                                acc, [r16, j * 16 + cols], v)
                return carry2
            lax.fori_loop(0, nck, _chunk, 0)
            return carry

        lax.fori_loop(0, nblk, _blk, 0)

        @pl.when(wid < 31)
        def _():
            pltpu.sync_copy(acc.at[pl.ds(0, _RT)],
                            out_hbm.at[pl.ds(base_row, _RT)])

        @pl.when(wid == 31)
        def _():
            pltpu.sync_copy(acc.at[pl.ds(0, _RT_LAST)],
                            out_hbm.at[pl.ds(base_row, _RT_LAST)])

    return seg


_seg_sum_cache = {}


def _seg_sum(depth):
    if depth not in _seg_sum_cache:
        _seg_sum_cache[depth] = _make_seg_sum(depth)
    return _seg_sum_cache[depth]


def _preproc():
    if "prep" not in _seg_sum_cache:
        _seg_sum_cache["prep"] = _make_preproc()
    return _seg_sum_cache["prep"]


_BM = 1000
_NB = _N // _BM
_NF = float(_N)


def _bn(x, s_ref, q_ref, g_ref, b_ref):
    m = s_ref[0:1, :] * (1.0 / _NF)
    v = q_ref[0:1, :] * (1.0 / _NF) - m * m
    r = lax.rsqrt(v + 1e-5)
    return (x - m) * r * g_ref[...] + b_ref[...]


def _stats_update(i, x, s_ref, q_ref):
    ssum = jnp.broadcast_to(jnp.sum(x, axis=0, keepdims=True), (8, _HID))
    qsum = jnp.broadcast_to(jnp.sum(x * x, axis=0, keepdims=True), (8, _HID))

    @pl.when(i == 0)
    def _():
        s_ref[...] = ssum
        q_ref[...] = qsum

    @pl.when(i > 0)
    def _():
        s_ref[...] += ssum
        q_ref[...] += qsum


def _row_spec(cols):
    return pl.BlockSpec((_BM, cols), lambda i: (i, 0))


def _fixed_spec(shape):
    return pl.BlockSpec(shape, lambda i: (0, 0))


_STATS_SHAPES = [jax.ShapeDtypeStruct((8, _HID), jnp.float32)] * 2
_STATS_SPECS = [_fixed_spec((8, _HID))] * 2


def _make_mm_stats(din):
    def body(scale_ref, h_ref, a_ref, w_ref, b_ref, u_ref, s_ref, q_ref):
        i = pl.program_id(0)
        t = scale_ref[0, 0] * h_ref[...] + a_ref[...]
        u = jnp.dot(t, w_ref[...], preferred_element_type=jnp.float32)
        u = u + b_ref[...]
        u_ref[...] = u
        _stats_update(i, u, s_ref, q_ref)

    return pl.pallas_call(
        body,
        grid=(_NB,),
        in_specs=[
            pl.BlockSpec(memory_space=pltpu.SMEM),
            _row_spec(din),
            _row_spec(din),
            _fixed_spec((din, _HID)),
            _fixed_spec((1, _HID)),
        ],
        out_specs=[_row_spec(_HID)] + _STATS_SPECS,
        out_shape=[jax.ShapeDtypeStruct((_N, _HID), jnp.float32)]
        + _STATS_SHAPES,
    )


_mm_stats_hid = _make_mm_stats(_HID)


def _bnrelu_mm_body(u_ref, s_ref, q_ref, g_ref, b_ref, w_ref, b2_ref,
                    v_ref, s2_ref, q2_ref):
    i = pl.program_id(0)
    x = jnp.maximum(_bn(u_ref[...], s_ref, q_ref, g_ref, b_ref), 0.0)
    v = jnp.dot(x, w_ref[...], preferred_element_type=jnp.float32)
    v = v + b2_ref[...]
    v_ref[...] = v
    _stats_update(i, v, s2_ref, q2_ref)


_bnrelu_mm = pl.pallas_call(
    _bnrelu_mm_body,
    grid=(_NB,),
    in_specs=[
        _row_spec(_HID),
        _fixed_spec((8, _HID)),
        _fixed_spec((8, _HID)),
        _fixed_spec((1, _HID)),
        _fixed_spec((1, _HID)),
        _fixed_spec((_HID, _HID)),
        _fixed_spec((1, _HID)),
    ],
    out_specs=[_row_spec(_HID)] + _STATS_SPECS,
    out_shape=[jax.ShapeDtypeStruct((_N, _HID), jnp.float32)] + _STATS_SHAPES,
)


def _bnrelu_scale_body(v_ref, s_ref, q_ref, g_ref, b_ref, sn_ref,
                       z_ref, s2_ref, q2_ref):
    i = pl.program_id(0)
    z = jnp.maximum(_bn(v_ref[...], s_ref, q_ref, g_ref, b_ref), 0.0)
    z = z * sn_ref[...]
    z_ref[...] = z
    _stats_update(i, z, s2_ref, q2_ref)


_bnrelu_scale = pl.pallas_call(
    _bnrelu_scale_body,
    grid=(_NB,),
    in_specs=[
        _row_spec(_HID),
        _fixed_spec((8, _HID)),
        _fixed_spec((8, _HID)),
        _fixed_spec((1, _HID)),
        _fixed_spec((1, _HID)),
        pl.BlockSpec((_BM, 1), lambda i: (i, 0)),
    ],
    out_specs=[_row_spec(_HID)] + _STATS_SPECS,
    out_shape=[jax.ShapeDtypeStruct((_N, _HID), jnp.float32)] + _STATS_SHAPES,
)


def _bnrelu_res_body(z_ref, s_ref, q_ref, g_ref, b_ref, hin_ref, o_ref):
    o = jnp.maximum(_bn(z_ref[...], s_ref, q_ref, g_ref, b_ref), 0.0)
    o_ref[...] = o + hin_ref[...]


_bnrelu_res = pl.pallas_call(
    _bnrelu_res_body,
    grid=(_NB,),
    in_specs=[
        _row_spec(_HID),
        _fixed_spec((8, _HID)),
        _fixed_spec((8, _HID)),
        _fixed_spec((1, _HID)),
        _fixed_spec((1, _HID)),
        _row_spec(_HID),
    ],
    out_specs=_row_spec(_HID),
    out_shape=jax.ShapeDtypeStruct((_N, _HID), jnp.float32),
)


def _bnrelu_body(z_ref, s_ref, q_ref, g_ref, b_ref, o_ref):
    o_ref[...] = jnp.maximum(_bn(z_ref[...], s_ref, q_ref, g_ref, b_ref), 0.0)


_bnrelu = pl.pallas_call(
    _bnrelu_body,
    grid=(_NB,),
    in_specs=[
        _row_spec(_HID),
        _fixed_spec((8, _HID)),
        _fixed_spec((8, _HID)),
        _fixed_spec((1, _HID)),
        _fixed_spec((1, _HID)),
    ],
    out_specs=_row_spec(_HID),
    out_shape=jax.ShapeDtypeStruct((_N, _HID), jnp.float32),
)


def _vec(p):
    return p.reshape(1, _HID)


def kernel(h, e, snorm_n, snorm_e, edge_index, params):
    csrc, cldst, cnts = _preproc()(edge_index[0], edge_index[1])
    for i in range(_L):
        p = params[i]
        h_in = h
        scale = jnp.reshape(1.0 + p["eps"], (1, 1))
        if i == 0:
            hp = jnp.concatenate(
                [h, jnp.zeros((_N, _HID - _IN_DIM), jnp.float32)], axis=1)
            w0p = jnp.concatenate(
                [p["W0"], jnp.zeros((_HID - _IN_DIM, _HID), jnp.float32)],
                axis=0)
            agg = _seg_sum(_HID)(hp, csrc, cldst, cnts)
            u, s1, q1 = _mm_stats_hid(scale, hp, agg, w0p, _vec(p["b0"]))
        else:
            agg = _seg_sum(_HID)(h, csrc, cldst, cnts)
            u0, s0, q0 = _mm_stats_hid(scale, h, agg, p["W1"], _vec(p["b1"]))
            u, s1, q1 = _bnrelu_mm(u0, s0, q0, _vec(p["g_mid"]),
                                   _vec(p["b_mid"]), p["W2"], _vec(p["b2"]))
        z, s2, q2 = _bnrelu_scale(u, s1, q1, _vec(p["g_app"]),
                                  _vec(p["b_app"]), snorm_n)
        if i == 0:
            h = _bnrelu(z, s2, q2, _vec(p["g_node"]), _vec(p["b_node"]))
        else:
            h = _bnrelu_res(z, s2, q2, _vec(p["g_node"]), _vec(p["b_node"]),
                            h_in)
    return h


# speedup vs baseline: 1.8865x; 1.1325x over previous
import functools

import jax
import jax.numpy as jnp
from jax import lax
from jax.experimental import pallas as pl
from jax.experimental.pallas import tpu as pltpu
from jax.experimental.pallas import tpu_sc as plsc

_N = 10000
_E = 320000
_IN_DIM = 128
_HID = 256
_L = 4


_CAP = 32768
_RT = 312
_RT_LAST = _N - 31 * _RT
_RPAD = 336
_DUMMY = 328
_IBLK = 8000
_NBLK = _E // _IBLK
_SB = 2048
_CK = 64
_CP = pltpu.CompilerParams(needs_layout_passes=False)


def _make_preproc():
    mesh = plsc.VectorSubcoreMesh(core_axis_name="c", subcore_axis_name="s")

    @functools.partial(
        pl.kernel, mesh=mesh,
        out_type=[jax.ShapeDtypeStruct((32, _CAP), jnp.int32),
                  jax.ShapeDtypeStruct((32, _CAP), jnp.int32),
                  jax.ShapeDtypeStruct((32, 16), jnp.int32)],
        compiler_params=_CP,
        scratch_types=[
            pltpu.VMEM((_IBLK,), jnp.int32),
            pltpu.VMEM((_IBLK,), jnp.int32),
            pltpu.VMEM((_CAP,), jnp.int32),
            pltpu.VMEM((_CAP,), jnp.int32),
            pltpu.VMEM((16,), jnp.int32),
        ],
    )
    def prep(src_hbm, dst_hbm, csrc_hbm, cldst_hbm, cnt_hbm,
             sblk, dblk, cbs, cbl, cnt_v):
        c = lax.axis_index("c")
        s = lax.axis_index("s")
        wid = s * 2 + c
        lo = wid * _RT
        hi = jnp.where(wid == 31, _N, lo + _RT)

        def _init(i, carry):
            cbs[pl.ds(i * 16, 16)] = jnp.zeros((16,), jnp.int32)
            cbl[pl.ds(i * 16, 16)] = jnp.full((16,), _DUMMY, jnp.int32)
            return carry
        lax.fori_loop(0, _CAP // 16, _init, 0)

        def _blk(b, ptr):
            pltpu.sync_copy(src_hbm.at[pl.ds(b * _IBLK, _IBLK)], sblk)
            pltpu.sync_copy(dst_hbm.at[pl.ds(b * _IBLK, _IBLK)], dblk)

            def _grp(g, ptr2):
                sv = sblk[pl.ds(g * 16, 16)]
                d = dblk[pl.ds(g * 16, 16)]
                m = (d >= lo) & (d < hi)
                plsc.store_compressed(cbs.at[pl.ds(ptr2, 16)], sv, mask=m)
                plsc.store_compressed(cbl.at[pl.ds(ptr2, 16)], d - lo, mask=m)
                pc = plsc.all_reduce_population_count(m)
                pc_s = pc if pc.ndim == 0 else pc[0]
                return ptr2 + pc_s
            return lax.fori_loop(0, _IBLK // 16, _grp, ptr)

        ptr = lax.fori_loop(0, _NBLK, _blk, 0)
        cnt_v[...] = jnp.zeros((16,), jnp.int32) + ptr
        pltpu.sync_copy(cbs, csrc_hbm.at[wid])
        pltpu.sync_copy(cbl, cldst_hbm.at[wid])
        pltpu.sync_copy(cnt_v, cnt_hbm.at[wid])

    return prep


def _make_seg_sum(depth):
    mesh = plsc.VectorSubcoreMesh(core_axis_name="c", subcore_axis_name="s")

    @functools.partial(
        pl.kernel, mesh=mesh,
        out_type=jax.ShapeDtypeStruct((_N, depth), jnp.float32),
        compiler_params=_CP,
        scratch_types=[
            pltpu.VMEM((_SB,), jnp.int32),
            pltpu.VMEM((_SB,), jnp.int32),
            pltpu.VMEM((_CK, depth), jnp.float32),
            pltpu.VMEM((_CK, depth), jnp.float32),
            pltpu.VMEM((_RPAD, depth), jnp.float32),
            pltpu.VMEM((16,), jnp.int32),
            pltpu.SemaphoreType.DMA,
            pltpu.SemaphoreType.DMA,
        ],
    )
    def seg(h_hbm, csrc_hbm, cldst_hbm, cnt_hbm, out_hbm,
            sblk, lblk, rows_a, rows_b, acc, cnt_v, sem_a, sem_b):
        c = lax.axis_index("c")
        s = lax.axis_index("s")
        wid = s * 2 + c
        base_row = wid * _RT
        ckmax = _SB // _CK

        def _zero(i, carry):
            for j in range(depth // 16):
                acc[i, pl.ds(j * 16, 16)] = jnp.zeros((16,), jnp.float32)
            return carry
        lax.fori_loop(0, _RPAD, _zero, 0)

        pltpu.sync_copy(cnt_hbm.at[wid], cnt_v)
        cnt = jnp.max(cnt_v[...])
        nblk = (cnt + _SB - 1) // _SB
        cols = lax.iota(jnp.int32, 16)

        def _gather(k, buf, sem):
            kc = jnp.minimum(k, ckmax - 1)
            return pltpu.async_copy(
                h_hbm.at[sblk.at[pl.ds(kc * _CK, _CK)]], buf, sem)

        def _process(k, buf):
            for g in range(_CK // 16):
                ld16 = lblk[pl.ds(k * _CK + g * 16, 16)]
                ok = (ld16 >= 0) & (ld16 < _DUMMY)
                ld16 = jnp.where(ok, ld16, _DUMMY)
                for el in range(16):
                    sel = jnp.zeros((16,), jnp.int32) + el
                    r16 = ld16.at[sel].get(mode="promise_in_bounds")
                    e = g * 16 + el
                    for j in range(depth // 16):
                        v = buf[e, pl.ds(j * 16, 16)]
                        plsc.addupdate_scatter(
                            acc, [r16, j * 16 + cols], v)

        def _blk(b, carry):
            pltpu.sync_copy(csrc_hbm.at[wid, pl.ds(b * _SB, _SB)], sblk)
            pltpu.sync_copy(cldst_hbm.at[wid, pl.ds(b * _SB, _SB)], lblk)
            rem = cnt - b * _SB
            nck = jnp.clip((rem + _CK - 1) // _CK, 0, ckmax)
            npair = (nck + 1) // 2
            _gather(0, rows_a, sem_a)

            def _pair(p, carry2):
                k0 = 2 * p
                _gather(k0 + 1, rows_b, sem_b)
                pltpu.make_async_copy(
                    h_hbm.at[sblk.at[pl.ds(0, _CK)]], rows_a, sem_a).wait()
                _process(k0, rows_a)
                _gather(k0 + 2, rows_a, sem_a)
                pltpu.make_async_copy(
                    h_hbm.at[sblk.at[pl.ds(0, _CK)]], rows_b, sem_b).wait()
                _process(k0 + 1, rows_b)
                return carry2

            lax.fori_loop(0, npair, _pair, 0)
            pltpu.make_async_copy(
                h_hbm.at[sblk.at[pl.ds(0, _CK)]], rows_a, sem_a).wait()
            return carry

        lax.fori_loop(0, nblk, _blk, 0)

        @pl.when(wid < 31)
        def _():
            pltpu.sync_copy(acc.at[pl.ds(0, _RT)],
                            out_hbm.at[pl.ds(base_row, _RT)])

        @pl.when(wid == 31)
        def _():
            pltpu.sync_copy(acc.at[pl.ds(0, _RT_LAST)],
                            out_hbm.at[pl.ds(base_row, _RT_LAST)])

    return seg


_seg_sum_cache = {}


def _seg_sum(depth):
    if depth not in _seg_sum_cache:
        _seg_sum_cache[depth] = _make_seg_sum(depth)
    return _seg_sum_cache[depth]


def _preproc():
    if "prep" not in _seg_sum_cache:
        _seg_sum_cache["prep"] = _make_preproc()
    return _seg_sum_cache["prep"]


_BM = 1000
_NB = _N // _BM
_NF = float(_N)


def _bn(x, s_ref, q_ref, g_ref, b_ref):
    m = s_ref[0:1, :] * (1.0 / _NF)
    v = q_ref[0:1, :] * (1.0 / _NF) - m * m
    r = lax.rsqrt(v + 1e-5)
    return (x - m) * r * g_ref[...] + b_ref[...]


def _stats_update(i, x, s_ref, q_ref):
    ssum = jnp.broadcast_to(jnp.sum(x, axis=0, keepdims=True), (8, _HID))
    qsum = jnp.broadcast_to(jnp.sum(x * x, axis=0, keepdims=True), (8, _HID))

    @pl.when(i == 0)
    def _():
        s_ref[...] = ssum
        q_ref[...] = qsum

    @pl.when(i > 0)
    def _():
        s_ref[...] += ssum
        q_ref[...] += qsum


def _row_spec(cols):
    return pl.BlockSpec((_BM, cols), lambda i: (i, 0))


def _fixed_spec(shape):
    return pl.BlockSpec(shape, lambda i: (0, 0))


_STATS_SHAPES = [jax.ShapeDtypeStruct((8, _HID), jnp.float32)] * 2
_STATS_SPECS = [_fixed_spec((8, _HID))] * 2


def _make_mm_stats(din):
    def body(scale_ref, h_ref, a_ref, w_ref, b_ref, u_ref, s_ref, q_ref):
        i = pl.program_id(0)
        t = scale_ref[0, 0] * h_ref[...] + a_ref[...]
        u = jnp.dot(t, w_ref[...], preferred_element_type=jnp.float32)
        u = u + b_ref[...]
        u_ref[...] = u
        _stats_update(i, u, s_ref, q_ref)

    return pl.pallas_call(
        body,
        grid=(_NB,),
        in_specs=[
            pl.BlockSpec(memory_space=pltpu.SMEM),
            _row_spec(din),
            _row_spec(din),
            _fixed_spec((din, _HID)),
            _fixed_spec((1, _HID)),
        ],
        out_specs=[_row_spec(_HID)] + _STATS_SPECS,
        out_shape=[jax.ShapeDtypeStruct((_N, _HID), jnp.float32)]
        + _STATS_SHAPES,
    )


_mm_stats_hid = _make_mm_stats(_HID)


def _bnrelu_mm_body(u_ref, s_ref, q_ref, g_ref, b_ref, w_ref, b2_ref,
                    v_ref, s2_ref, q2_ref):
    i = pl.program_id(0)
    x = jnp.maximum(_bn(u_ref[...], s_ref, q_ref, g_ref, b_ref), 0.0)
    v = jnp.dot(x, w_ref[...], preferred_element_type=jnp.float32)
    v = v + b2_ref[...]
    v_ref[...] = v
    _stats_update(i, v, s2_ref, q2_ref)


_bnrelu_mm = pl.pallas_call(
    _bnrelu_mm_body,
    grid=(_NB,),
    in_specs=[
        _row_spec(_HID),
        _fixed_spec((8, _HID)),
        _fixed_spec((8, _HID)),
        _fixed_spec((1, _HID)),
        _fixed_spec((1, _HID)),
        _fixed_spec((_HID, _HID)),
        _fixed_spec((1, _HID)),
    ],
    out_specs=[_row_spec(_HID)] + _STATS_SPECS,
    out_shape=[jax.ShapeDtypeStruct((_N, _HID), jnp.float32)] + _STATS_SHAPES,
)


def _bnrelu_scale_body(v_ref, s_ref, q_ref, g_ref, b_ref, sn_ref,
                       z_ref, s2_ref, q2_ref):
    i = pl.program_id(0)
    z = jnp.maximum(_bn(v_ref[...], s_ref, q_ref, g_ref, b_ref), 0.0)
    z = z * sn_ref[...]
    z_ref[...] = z
    _stats_update(i, z, s2_ref, q2_ref)


_bnrelu_scale = pl.pallas_call(
    _bnrelu_scale_body,
    grid=(_NB,),
    in_specs=[
        _row_spec(_HID),
        _fixed_spec((8, _HID)),
        _fixed_spec((8, _HID)),
        _fixed_spec((1, _HID)),
        _fixed_spec((1, _HID)),
        pl.BlockSpec((_BM, 1), lambda i: (i, 0)),
    ],
    out_specs=[_row_spec(_HID)] + _STATS_SPECS,
    out_shape=[jax.ShapeDtypeStruct((_N, _HID), jnp.float32)] + _STATS_SHAPES,
)


def _bnrelu_res_body(z_ref, s_ref, q_ref, g_ref, b_ref, hin_ref, o_ref):
    o = jnp.maximum(_bn(z_ref[...], s_ref, q_ref, g_ref, b_ref), 0.0)
    o_ref[...] = o + hin_ref[...]


_bnrelu_res = pl.pallas_call(
    _bnrelu_res_body,
    grid=(_NB,),
    in_specs=[
        _row_spec(_HID),
        _fixed_spec((8, _HID)),
        _fixed_spec((8, _HID)),
        _fixed_spec((1, _HID)),
        _fixed_spec((1, _HID)),
        _row_spec(_HID),
    ],
    out_specs=_row_spec(_HID),
    out_shape=jax.ShapeDtypeStruct((_N, _HID), jnp.float32),
)


def _bnrelu_body(z_ref, s_ref, q_ref, g_ref, b_ref, o_ref):
    o_ref[...] = jnp.maximum(_bn(z_ref[...], s_ref, q_ref, g_ref, b_ref), 0.0)


_bnrelu = pl.pallas_call(
    _bnrelu_body,
    grid=(_NB,),
    in_specs=[
        _row_spec(_HID),
        _fixed_spec((8, _HID)),
        _fixed_spec((8, _HID)),
        _fixed_spec((1, _HID)),
        _fixed_spec((1, _HID)),
    ],
    out_specs=_row_spec(_HID),
    out_shape=jax.ShapeDtypeStruct((_N, _HID), jnp.float32),
)


def _vec(p):
    return p.reshape(1, _HID)


def kernel(h, e, snorm_n, snorm_e, edge_index, params):
    csrc, cldst, cnts = _preproc()(edge_index[0], edge_index[1])
    for i in range(_L):
        p = params[i]
        h_in = h
        scale = jnp.reshape(1.0 + p["eps"], (1, 1))
        if i == 0:
            hp = jnp.concatenate(
                [h, jnp.zeros((_N, _HID - _IN_DIM), jnp.float32)], axis=1)
            w0p = jnp.concatenate(
                [p["W0"], jnp.zeros((_HID - _IN_DIM, _HID), jnp.float32)],
                axis=0)
            agg = _seg_sum(_HID)(hp, csrc, cldst, cnts)
            u, s1, q1 = _mm_stats_hid(scale, hp, agg, w0p, _vec(p["b0"]))
        else:
            agg = _seg_sum(_HID)(h, csrc, cldst, cnts)
            u0, s0, q0 = _mm_stats_hid(scale, h, agg, p["W1"], _vec(p["b1"]))
            u, s1, q1 = _bnrelu_mm(u0, s0, q0, _vec(p["g_mid"]),
                                   _vec(p["b_mid"]), p["W2"], _vec(p["b2"]))
        z, s2, q2 = _bnrelu_scale(u, s1, q1, _vec(p["g_app"]),
                                  _vec(p["b_app"]), snorm_n)
        if i == 0:
            h = _bnrelu(z, s2, q2, _vec(p["g_node"]), _vec(p["b_node"]))
        else:
            h = _bnrelu_res(z, s2, q2, _vec(p["g_node"]), _vec(p["b_node"]),
                            h_in)
    return h


# speedup vs baseline: 2.0902x; 1.1080x over previous
import functools

import jax
import jax.numpy as jnp
from jax import lax
from jax.experimental import pallas as pl
from jax.experimental.pallas import tpu as pltpu
from jax.experimental.pallas import tpu_sc as plsc

_N = 10000
_E = 320000
_IN_DIM = 128
_HID = 256
_L = 4


_CAP = 32768
_RT = 312
_RT_LAST = _N - 31 * _RT
_RPAD = 336
_DUMMY = 328
_IBLK = 8000
_NBLK = _E // _IBLK
_SB = 2048
_CK = 64
_CP = pltpu.CompilerParams(needs_layout_passes=False)


def _make_preproc():
    mesh = plsc.VectorSubcoreMesh(core_axis_name="c", subcore_axis_name="s")

    @functools.partial(
        pl.kernel, mesh=mesh,
        out_type=[jax.ShapeDtypeStruct((32, _CAP), jnp.int32),
                  jax.ShapeDtypeStruct((32, _CAP), jnp.int32),
                  jax.ShapeDtypeStruct((32, 16), jnp.int32)],
        compiler_params=_CP,
        scratch_types=[
            pltpu.VMEM((_IBLK,), jnp.int32),
            pltpu.VMEM((_IBLK,), jnp.int32),
            pltpu.VMEM((_CAP,), jnp.int32),
            pltpu.VMEM((_CAP,), jnp.int32),
            pltpu.VMEM((16,), jnp.int32),
        ],
    )
    def prep(src_hbm, dst_hbm, csrc_hbm, cldst_hbm, cnt_hbm,
             sblk, dblk, cbs, cbl, cnt_v):
        c = lax.axis_index("c")
        s = lax.axis_index("s")
        wid = s * 2 + c
        lo = wid * _RT
        hi = jnp.where(wid == 31, _N, lo + _RT)

        def _init(i, carry):
            cbs[pl.ds(i * 16, 16)] = jnp.zeros((16,), jnp.int32)
            cbl[pl.ds(i * 16, 16)] = jnp.full((16,), _DUMMY, jnp.int32)
            return carry
        lax.fori_loop(0, _CAP // 16, _init, 0)

        def _blk(b, ptr):
            pltpu.sync_copy(src_hbm.at[pl.ds(b * _IBLK, _IBLK)], sblk)
            pltpu.sync_copy(dst_hbm.at[pl.ds(b * _IBLK, _IBLK)], dblk)

            def _grp(g, ptr2):
                sv = sblk[pl.ds(g * 16, 16)]
                d = dblk[pl.ds(g * 16, 16)]
                m = (d >= lo) & (d < hi)
                plsc.store_compressed(cbs.at[pl.ds(ptr2, 16)], sv, mask=m)
                plsc.store_compressed(cbl.at[pl.ds(ptr2, 16)], d - lo, mask=m)
                pc = plsc.all_reduce_population_count(m)
                pc_s = pc if pc.ndim == 0 else pc[0]
                return ptr2 + pc_s
            return lax.fori_loop(0, _IBLK // 16, _grp, ptr)

        ptr = lax.fori_loop(0, _NBLK, _blk, 0)
        cnt_v[...] = jnp.zeros((16,), jnp.int32) + ptr
        pltpu.sync_copy(cbs, csrc_hbm.at[wid])
        pltpu.sync_copy(cbl, cldst_hbm.at[wid])
        pltpu.sync_copy(cnt_v, cnt_hbm.at[wid])

    return prep


def _make_seg_sum(depth):
    mesh = plsc.VectorSubcoreMesh(core_axis_name="c", subcore_axis_name="s")

    @functools.partial(
        pl.kernel, mesh=mesh,
        out_type=jax.ShapeDtypeStruct((_N * depth,), jnp.float32),
        compiler_params=_CP,
        scratch_types=[
            pltpu.VMEM((_SB,), jnp.int32),
            pltpu.VMEM((_SB,), jnp.int32),
            pltpu.VMEM((_CK, depth), jnp.float32),
            pltpu.VMEM((_CK, depth), jnp.float32),
            pltpu.VMEM((_RPAD * depth,), jnp.float32),
            pltpu.VMEM((16,), jnp.int32),
            pltpu.SemaphoreType.DMA,
            pltpu.SemaphoreType.DMA,
        ],
    )
    def seg(h_hbm, csrc_hbm, cldst_hbm, cnt_hbm, out_hbm,
            sblk, lblk, rows_a, rows_b, acc, cnt_v, sem_a, sem_b):
        c = lax.axis_index("c")
        s = lax.axis_index("s")
        wid = s * 2 + c
        base_row = wid * _RT
        ckmax = _SB // _CK

        def _zero(i, carry):
            for j in range(depth):
                acc[pl.ds((i * depth + j) * 16, 16)] = jnp.zeros(
                    (16,), jnp.float32)
            return carry
        lax.fori_loop(0, _RPAD * 16 // 16 // 16, _zero, 0)

        pltpu.sync_copy(cnt_hbm.at[wid], cnt_v)
        cnt = jnp.max(cnt_v[...])
        nblk = (cnt + _SB - 1) // _SB
        cols = lax.iota(jnp.int32, 16)

        def _gather(k, buf, sem):
            kc = jnp.minimum(k, ckmax - 1)
            return pltpu.async_copy(
                h_hbm.at[sblk.at[pl.ds(kc * _CK, _CK)]], buf, sem)

        def _process(k, buf):
            for g in range(_CK // 16):
                ld16 = lblk[pl.ds(k * _CK + g * 16, 16)]
                ok = (ld16 >= 0) & (ld16 < _DUMMY)
                base16 = jnp.where(ok, ld16, _DUMMY) * depth
                for el in range(16):
                    sel = jnp.zeros((16,), jnp.int32) + el
                    bse = base16.at[sel].get(mode="promise_in_bounds")
                    e = g * 16 + el
                    for j in range(depth // 16):
                        v = buf[e, pl.ds(j * 16, 16)]
                        plsc.addupdate_scatter(
                            acc, [bse + (cols + j * 16)], v)

        def _blk(b, carry):
            pltpu.sync_copy(csrc_hbm.at[wid, pl.ds(b * _SB, _SB)], sblk)
            pltpu.sync_copy(cldst_hbm.at[wid, pl.ds(b * _SB, _SB)], lblk)
            rem = cnt - b * _SB
            nck = jnp.clip((rem + _CK - 1) // _CK, 0, ckmax)
            npair = (nck + 1) // 2
            _gather(0, rows_a, sem_a)

            def _pair(p, carry2):
                k0 = 2 * p
                _gather(k0 + 1, rows_b, sem_b)
                pltpu.make_async_copy(
                    h_hbm.at[sblk.at[pl.ds(0, _CK)]], rows_a, sem_a).wait()
                _process(k0, rows_a)
                _gather(k0 + 2, rows_a, sem_a)
                pltpu.make_async_copy(
                    h_hbm.at[sblk.at[pl.ds(0, _CK)]], rows_b, sem_b).wait()
                _process(k0 + 1, rows_b)
                return carry2

            lax.fori_loop(0, npair, _pair, 0)
            pltpu.make_async_copy(
                h_hbm.at[sblk.at[pl.ds(0, _CK)]], rows_a, sem_a).wait()
            return carry

        lax.fori_loop(0, nblk, _blk, 0)

        @pl.when(wid < 31)
        def _():
            pltpu.sync_copy(acc.at[pl.ds(0, _RT * depth)],
                            out_hbm.at[pl.ds(base_row * depth, _RT * depth)])

        @pl.when(wid == 31)
        def _():
            pltpu.sync_copy(
                acc.at[pl.ds(0, _RT_LAST * depth)],
                out_hbm.at[pl.ds(base_row * depth, _RT_LAST * depth)])

    return seg


_seg_sum_cache = {}


def _seg_sum(depth):
    if depth not in _seg_sum_cache:
        _seg_sum_cache[depth] = _make_seg_sum(depth)
    return _seg_sum_cache[depth]


def _preproc():
    if "prep" not in _seg_sum_cache:
        _seg_sum_cache["prep"] = _make_preproc()
    return _seg_sum_cache["prep"]


_BM = 1000
_NB = _N // _BM
_NF = float(_N)


def _bn(x, s_ref, q_ref, g_ref, b_ref):
    m = s_ref[0:1, :] * (1.0 / _NF)
    v = q_ref[0:1, :] * (1.0 / _NF) - m * m
    r = lax.rsqrt(v + 1e-5)
    return (x - m) * r * g_ref[...] + b_ref[...]


def _stats_update(i, x, s_ref, q_ref):
    ssum = jnp.broadcast_to(jnp.sum(x, axis=0, keepdims=True), (8, _HID))
    qsum = jnp.broadcast_to(jnp.sum(x * x, axis=0, keepdims=True), (8, _HID))

    @pl.when(i == 0)
    def _():
        s_ref[...] = ssum
        q_ref[...] = qsum

    @pl.when(i > 0)
    def _():
        s_ref[...] += ssum
        q_ref[...] += qsum


def _row_spec(cols):
    return pl.BlockSpec((_BM, cols), lambda i: (i, 0))


def _fixed_spec(shape):
    return pl.BlockSpec(shape, lambda i: (0, 0))


_STATS_SHAPES = [jax.ShapeDtypeStruct((8, _HID), jnp.float32)] * 2
_STATS_SPECS = [_fixed_spec((8, _HID))] * 2


def _make_mm_stats(din):
    def body(scale_ref, h_ref, a_ref, w_ref, b_ref, u_ref, s_ref, q_ref):
        i = pl.program_id(0)
        t = scale_ref[0, 0] * h_ref[...] + a_ref[...]
        u = jnp.dot(t, w_ref[...], preferred_element_type=jnp.float32)
        u = u + b_ref[...]
        u_ref[...] = u
        _stats_update(i, u, s_ref, q_ref)

    return pl.pallas_call(
        body,
        grid=(_NB,),
        in_specs=[
            pl.BlockSpec(memory_space=pltpu.SMEM),
            _row_spec(din),
            _row_spec(din),
            _fixed_spec((din, _HID)),
            _fixed_spec((1, _HID)),
        ],
        out_specs=[_row_spec(_HID)] + _STATS_SPECS,
        out_shape=[jax.ShapeDtypeStruct((_N, _HID), jnp.float32)]
        + _STATS_SHAPES,
    )


_mm_stats_in = _make_mm_stats(_IN_DIM)
_mm_stats_hid = _make_mm_stats(_HID)


def _bnrelu_mm_body(u_ref, s_ref, q_ref, g_ref, b_ref, w_ref, b2_ref,
                    v_ref, s2_ref, q2_ref):
    i = pl.program_id(0)
    x = jnp.maximum(_bn(u_ref[...], s_ref, q_ref, g_ref, b_ref), 0.0)
    v = jnp.dot(x, w_ref[...], preferred_element_type=jnp.float32)
    v = v + b2_ref[...]
    v_ref[...] = v
    _stats_update(i, v, s2_ref, q2_ref)


_bnrelu_mm = pl.pallas_call(
    _bnrelu_mm_body,
    grid=(_NB,),
    in_specs=[
        _row_spec(_HID),
        _fixed_spec((8, _HID)),
        _fixed_spec((8, _HID)),
        _fixed_spec((1, _HID)),
        _fixed_spec((1, _HID)),
        _fixed_spec((_HID, _HID)),
        _fixed_spec((1, _HID)),
    ],
    out_specs=[_row_spec(_HID)] + _STATS_SPECS,
    out_shape=[jax.ShapeDtypeStruct((_N, _HID), jnp.float32)] + _STATS_SHAPES,
)


def _bnrelu_scale_body(v_ref, s_ref, q_ref, g_ref, b_ref, sn_ref,
                       z_ref, s2_ref, q2_ref):
    i = pl.program_id(0)
    z = jnp.maximum(_bn(v_ref[...], s_ref, q_ref, g_ref, b_ref), 0.0)
    z = z * sn_ref[...]
    z_ref[...] = z
    _stats_update(i, z, s2_ref, q2_ref)


_bnrelu_scale = pl.pallas_call(
    _bnrelu_scale_body,
    grid=(_NB,),
    in_specs=[
        _row_spec(_HID),
        _fixed_spec((8, _HID)),
        _fixed_spec((8, _HID)),
        _fixed_spec((1, _HID)),
        _fixed_spec((1, _HID)),
        pl.BlockSpec((_BM, 1), lambda i: (i, 0)),
    ],
    out_specs=[_row_spec(_HID)] + _STATS_SPECS,
    out_shape=[jax.ShapeDtypeStruct((_N, _HID), jnp.float32)] + _STATS_SHAPES,
)


def _bnrelu_res_body(z_ref, s_ref, q_ref, g_ref, b_ref, hin_ref, o_ref):
    o = jnp.maximum(_bn(z_ref[...], s_ref, q_ref, g_ref, b_ref), 0.0)
    o_ref[...] = o + hin_ref[...]


_bnrelu_res = pl.pallas_call(
    _bnrelu_res_body,
    grid=(_NB,),
    in_specs=[
        _row_spec(_HID),
        _fixed_spec((8, _HID)),
        _fixed_spec((8, _HID)),
        _fixed_spec((1, _HID)),
        _fixed_spec((1, _HID)),
        _row_spec(_HID),
    ],
    out_specs=_row_spec(_HID),
    out_shape=jax.ShapeDtypeStruct((_N, _HID), jnp.float32),
)


def _bnrelu_body(z_ref, s_ref, q_ref, g_ref, b_ref, o_ref):
    o_ref[...] = jnp.maximum(_bn(z_ref[...], s_ref, q_ref, g_ref, b_ref), 0.0)


_bnrelu = pl.pallas_call(
    _bnrelu_body,
    grid=(_NB,),
    in_specs=[
        _row_spec(_HID),
        _fixed_spec((8, _HID)),
        _fixed_spec((8, _HID)),
        _fixed_spec((1, _HID)),
        _fixed_spec((1, _HID)),
    ],
    out_specs=_row_spec(_HID),
    out_shape=jax.ShapeDtypeStruct((_N, _HID), jnp.float32),
)


def _vec(p):
    return p.reshape(1, _HID)


def kernel(h, e, snorm_n, snorm_e, edge_index, params):
    csrc, cldst, cnts = _preproc()(edge_index[0], edge_index[1])
    for i in range(_L):
        p = params[i]
        h_in = h
        scale = jnp.reshape(1.0 + p["eps"], (1, 1))
        if i == 0:
            agg = _seg_sum(_IN_DIM)(
                h, csrc, cldst, cnts).reshape(_N, _IN_DIM)
            u, s1, q1 = _mm_stats_in(scale, h, agg, p["W0"], _vec(p["b0"]))
        else:
            agg = _seg_sum(_HID)(
                h, csrc, cldst, cnts).reshape(_N, _HID)
            u0, s0, q0 = _mm_stats_hid(scale, h, agg, p["W1"], _vec(p["b1"]))
            u, s1, q1 = _bnrelu_mm(u0, s0, q0, _vec(p["g_mid"]),
                                   _vec(p["b_mid"]), p["W2"], _vec(p["b2"]))
        z, s2, q2 = _bnrelu_scale(u, s1, q1, _vec(p["g_app"]),
                                  _vec(p["b_app"]), snorm_n)
        if i == 0:
            h = _bnrelu(z, s2, q2, _vec(p["g_node"]), _vec(p["b_node"]))
        else:
            h = _bnrelu_res(z, s2, q2, _vec(p["g_node"]), _vec(p["b_node"]),
                            h_in)
    return h


# speedup vs baseline: 2.1086x; 1.0088x over previous
import functools

import jax
import jax.numpy as jnp
from jax import lax
from jax.experimental import pallas as pl
from jax.experimental.pallas import tpu as pltpu
from jax.experimental.pallas import tpu_sc as plsc

_N = 10000
_E = 320000
_IN_DIM = 128
_HID = 256
_L = 4


_CAP = 32768
_RT = 312
_RT_LAST = _N - 31 * _RT
_RPAD = 336
_DUMMY = 328
_IBLK = 8000
_NBLK = _E // _IBLK
_SB = 2048
_CK = 64
_CP = pltpu.CompilerParams(needs_layout_passes=False)


def _make_preproc():
    mesh = plsc.VectorSubcoreMesh(core_axis_name="c", subcore_axis_name="s")

    @functools.partial(
        pl.kernel, mesh=mesh,
        out_type=[jax.ShapeDtypeStruct((32, _CAP), jnp.int32),
                  jax.ShapeDtypeStruct((32, _CAP), jnp.int32),
                  jax.ShapeDtypeStruct((32, 16), jnp.int32)],
        compiler_params=_CP,
        scratch_types=[
            pltpu.VMEM((_IBLK,), jnp.int32),
            pltpu.VMEM((_IBLK,), jnp.int32),
            pltpu.VMEM((_CAP,), jnp.int32),
            pltpu.VMEM((_CAP,), jnp.int32),
            pltpu.VMEM((16,), jnp.int32),
        ],
    )
    def prep(src_hbm, dst_hbm, csrc_hbm, cldst_hbm, cnt_hbm,
             sblk, dblk, cbs, cbl, cnt_v):
        c = lax.axis_index("c")
        s = lax.axis_index("s")
        wid = s * 2 + c
        lo = wid * _RT
        hi = jnp.where(wid == 31, _N, lo + _RT)

        def _init(i, carry):
            cbs[pl.ds(i * 16, 16)] = jnp.zeros((16,), jnp.int32)
            cbl[pl.ds(i * 16, 16)] = jnp.full((16,), _DUMMY, jnp.int32)
            return carry
        lax.fori_loop(0, _CAP // 16, _init, 0)

        def _blk(b, ptr):
            pltpu.sync_copy(src_hbm.at[pl.ds(b * _IBLK, _IBLK)], sblk)
            pltpu.sync_copy(dst_hbm.at[pl.ds(b * _IBLK, _IBLK)], dblk)

            def _grp(g, ptr2):
                for u in range(4):
                    sv = sblk[pl.ds((g * 4 + u) * 16, 16)]
                    d = dblk[pl.ds((g * 4 + u) * 16, 16)]
                    m = (d >= lo) & (d < hi)
                    plsc.store_compressed(cbs.at[pl.ds(ptr2, 16)], sv,
                                          mask=m)
                    plsc.store_compressed(cbl.at[pl.ds(ptr2, 16)], d - lo,
                                          mask=m)
                    pc = plsc.all_reduce_population_count(m)
                    pc_s = pc if pc.ndim == 0 else pc[0]
                    ptr2 = ptr2 + pc_s
                return ptr2
            return lax.fori_loop(0, _IBLK // 64, _grp, ptr)

        ptr = lax.fori_loop(0, _NBLK, _blk, 0)
        cnt_v[...] = jnp.zeros((16,), jnp.int32) + ptr
        pltpu.sync_copy(cbs, csrc_hbm.at[wid])
        pltpu.sync_copy(cbl, cldst_hbm.at[wid])
        pltpu.sync_copy(cnt_v, cnt_hbm.at[wid])

    return prep


def _make_seg_sum(depth):
    mesh = plsc.VectorSubcoreMesh(core_axis_name="c", subcore_axis_name="s")

    @functools.partial(
        pl.kernel, mesh=mesh,
        out_type=jax.ShapeDtypeStruct((_N * depth,), jnp.float32),
        compiler_params=_CP,
        scratch_types=[
            pltpu.VMEM((_SB,), jnp.int32),
            pltpu.VMEM((_SB,), jnp.int32),
            pltpu.VMEM((_CK, depth), jnp.float32),
            pltpu.VMEM((_CK, depth), jnp.float32),
            pltpu.VMEM((_RPAD * depth,), jnp.float32),
            pltpu.VMEM((16,), jnp.int32),
            pltpu.SemaphoreType.DMA,
            pltpu.SemaphoreType.DMA,
        ],
    )
    def seg(h_hbm, csrc_hbm, cldst_hbm, cnt_hbm, out_hbm,
            sblk, lblk, rows_a, rows_b, acc, cnt_v, sem_a, sem_b):
        c = lax.axis_index("c")
        s = lax.axis_index("s")
        wid = s * 2 + c
        base_row = wid * _RT
        ckmax = _SB // _CK

        def _zero(i, carry):
            for j in range(depth):
                acc[pl.ds((i * depth + j) * 16, 16)] = jnp.zeros(
                    (16,), jnp.float32)
            return carry
        lax.fori_loop(0, _RPAD * 16 // 16 // 16, _zero, 0)

        pltpu.sync_copy(cnt_hbm.at[wid], cnt_v)
        cnt = jnp.max(cnt_v[...])
        nblk = (cnt + _SB - 1) // _SB
        cols = lax.iota(jnp.int32, 16)

        def _gather(k, buf, sem):
            kc = jnp.minimum(k, ckmax - 1)
            return pltpu.async_copy(
                h_hbm.at[sblk.at[pl.ds(kc * _CK, _CK)]], buf, sem)

        def _process(k, buf):
            for g in range(_CK // 16):
                ld16 = lblk[pl.ds(k * _CK + g * 16, 16)]
                ok = (ld16 >= 0) & (ld16 < _DUMMY)
                base16 = jnp.where(ok, ld16, _DUMMY) * depth
                for el in range(16):
                    sel = jnp.zeros((16,), jnp.int32) + el
                    bse = base16.at[sel].get(mode="promise_in_bounds")
                    e = g * 16 + el
                    for j in range(depth // 16):
                        v = buf[e, pl.ds(j * 16, 16)]
                        plsc.addupdate_scatter(
                            acc, [bse + (cols + j * 16)], v)

        def _blk(b, carry):
            pltpu.sync_copy(csrc_hbm.at[wid, pl.ds(b * _SB, _SB)], sblk)
            pltpu.sync_copy(cldst_hbm.at[wid, pl.ds(b * _SB, _SB)], lblk)
            rem = cnt - b * _SB
            nck = jnp.clip((rem + _CK - 1) // _CK, 0, ckmax)
            npair = (nck + 1) // 2
            _gather(0, rows_a, sem_a)

            def _pair(p, carry2):
                k0 = 2 * p
                _gather(k0 + 1, rows_b, sem_b)
                pltpu.make_async_copy(
                    h_hbm.at[sblk.at[pl.ds(0, _CK)]], rows_a, sem_a).wait()
                _process(k0, rows_a)
                _gather(k0 + 2, rows_a, sem_a)
                pltpu.make_async_copy(
                    h_hbm.at[sblk.at[pl.ds(0, _CK)]], rows_b, sem_b).wait()
                _process(k0 + 1, rows_b)
                return carry2

            lax.fori_loop(0, npair, _pair, 0)
            pltpu.make_async_copy(
                h_hbm.at[sblk.at[pl.ds(0, _CK)]], rows_a, sem_a).wait()
            return carry

        lax.fori_loop(0, nblk, _blk, 0)

        @pl.when(wid < 31)
        def _():
            pltpu.sync_copy(acc.at[pl.ds(0, _RT * depth)],
                            out_hbm.at[pl.ds(base_row * depth, _RT * depth)])

        @pl.when(wid == 31)
        def _():
            pltpu.sync_copy(
                acc.at[pl.ds(0, _RT_LAST * depth)],
                out_hbm.at[pl.ds(base_row * depth, _RT_LAST * depth)])

    return seg


_seg_sum_cache = {}


def _seg_sum(depth):
    if depth not in _seg_sum_cache:
        _seg_sum_cache[depth] = _make_seg_sum(depth)
    return _seg_sum_cache[depth]


def _preproc():
    if "prep" not in _seg_sum_cache:
        _seg_sum_cache["prep"] = _make_preproc()
    return _seg_sum_cache["prep"]


_BM = 1000
_NB = _N // _BM
_NF = float(_N)


def _bn(x, s_ref, q_ref, g_ref, b_ref):
    m = s_ref[0:1, :] * (1.0 / _NF)
    v = q_ref[0:1, :] * (1.0 / _NF) - m * m
    r = lax.rsqrt(v + 1e-5)
    return (x - m) * r * g_ref[...] + b_ref[...]


def _stats_update(i, x, s_ref, q_ref):
    ssum = jnp.broadcast_to(jnp.sum(x, axis=0, keepdims=True), (8, _HID))
    qsum = jnp.broadcast_to(jnp.sum(x * x, axis=0, keepdims=True), (8, _HID))

    @pl.when(i == 0)
    def _():
        s_ref[...] = ssum
        q_ref[...] = qsum

    @pl.when(i > 0)
    def _():
        s_ref[...] += ssum
        q_ref[...] += qsum


def _row_spec(cols):
    return pl.BlockSpec((_BM, cols), lambda i: (i, 0))


def _fixed_spec(shape):
    return pl.BlockSpec(shape, lambda i: (0, 0))


_STATS_SHAPES = [jax.ShapeDtypeStruct((8, _HID), jnp.float32)] * 2
_STATS_SPECS = [_fixed_spec((8, _HID))] * 2


def _make_mm_stats(din):
    def body(scale_ref, h_ref, a_ref, w_ref, b_ref, u_ref, s_ref, q_ref):
        i = pl.program_id(0)
        t = scale_ref[0, 0] * h_ref[...] + a_ref[...]
        u = jnp.dot(t, w_ref[...], preferred_element_type=jnp.float32)
        u = u + b_ref[...]
        u_ref[...] = u
        _stats_update(i, u, s_ref, q_ref)

    return pl.pallas_call(
        body,
        grid=(_NB,),
        in_specs=[
            pl.BlockSpec(memory_space=pltpu.SMEM),
            _row_spec(din),
            _row_spec(din),
            _fixed_spec((din, _HID)),
            _fixed_spec((1, _HID)),
        ],
        out_specs=[_row_spec(_HID)] + _STATS_SPECS,
        out_shape=[jax.ShapeDtypeStruct((_N, _HID), jnp.float32)]
        + _STATS_SHAPES,
    )


_mm_stats_in = _make_mm_stats(_IN_DIM)
_mm_stats_hid = _make_mm_stats(_HID)


def _bnrelu_mm_body(u_ref, s_ref, q_ref, g_ref, b_ref, w_ref, b2_ref,
                    v_ref, s2_ref, q2_ref):
    i = pl.program_id(0)
    x = jnp.maximum(_bn(u_ref[...], s_ref, q_ref, g_ref, b_ref), 0.0)
    v = jnp.dot(x, w_ref[...], preferred_element_type=jnp.float32)
    v = v + b2_ref[...]
    v_ref[...] = v
    _stats_update(i, v, s2_ref, q2_ref)


_bnrelu_mm = pl.pallas_call(
    _bnrelu_mm_body,
    grid=(_NB,),
    in_specs=[
        _row_spec(_HID),
        _fixed_spec((8, _HID)),
        _fixed_spec((8, _HID)),
        _fixed_spec((1, _HID)),
        _fixed_spec((1, _HID)),
        _fixed_spec((_HID, _HID)),
        _fixed_spec((1, _HID)),
    ],
    out_specs=[_row_spec(_HID)] + _STATS_SPECS,
    out_shape=[jax.ShapeDtypeStruct((_N, _HID), jnp.float32)] + _STATS_SHAPES,
)


def _bnrelu_scale_body(v_ref, s_ref, q_ref, g_ref, b_ref, sn_ref,
                       z_ref, s2_ref, q2_ref):
    i = pl.program_id(0)
    z = jnp.maximum(_bn(v_ref[...], s_ref, q_ref, g_ref, b_ref), 0.0)
    z = z * sn_ref[...]
    z_ref[...] = z
    _stats_update(i, z, s2_ref, q2_ref)


_bnrelu_scale = pl.pallas_call(
    _bnrelu_scale_body,
    grid=(_NB,),
    in_specs=[
        _row_spec(_HID),
        _fixed_spec((8, _HID)),
        _fixed_spec((8, _HID)),
        _fixed_spec((1, _HID)),
        _fixed_spec((1, _HID)),
        pl.BlockSpec((_BM, 1), lambda i: (i, 0)),
    ],
    out_specs=[_row_spec(_HID)] + _STATS_SPECS,
    out_shape=[jax.ShapeDtypeStruct((_N, _HID), jnp.float32)] + _STATS_SHAPES,
)


def _bnrelu_res_body(z_ref, s_ref, q_ref, g_ref, b_ref, hin_ref, o_ref):
    o = jnp.maximum(_bn(z_ref[...], s_ref, q_ref, g_ref, b_ref), 0.0)
    o_ref[...] = o + hin_ref[...]


_bnrelu_res = pl.pallas_call(
    _bnrelu_res_body,
    grid=(_NB,),
    in_specs=[
        _row_spec(_HID),
        _fixed_spec((8, _HID)),
        _fixed_spec((8, _HID)),
        _fixed_spec((1, _HID)),
        _fixed_spec((1, _HID)),
        _row_spec(_HID),
    ],
    out_specs=_row_spec(_HID),
    out_shape=jax.ShapeDtypeStruct((_N, _HID), jnp.float32),
)


def _bnrelu_body(z_ref, s_ref, q_ref, g_ref, b_ref, o_ref):
    o_ref[...] = jnp.maximum(_bn(z_ref[...], s_ref, q_ref, g_ref, b_ref), 0.0)


_bnrelu = pl.pallas_call(
    _bnrelu_body,
    grid=(_NB,),
    in_specs=[
        _row_spec(_HID),
        _fixed_spec((8, _HID)),
        _fixed_spec((8, _HID)),
        _fixed_spec((1, _HID)),
        _fixed_spec((1, _HID)),
    ],
    out_specs=_row_spec(_HID),
    out_shape=jax.ShapeDtypeStruct((_N, _HID), jnp.float32),
)


def _vec(p):
    return p.reshape(1, _HID)


def kernel(h, e, snorm_n, snorm_e, edge_index, params):
    csrc, cldst, cnts = _preproc()(edge_index[0], edge_index[1])
    for i in range(_L):
        p = params[i]
        h_in = h
        scale = jnp.reshape(1.0 + p["eps"], (1, 1))
        if i == 0:
            agg = _seg_sum(_IN_DIM)(
                h, csrc, cldst, cnts).reshape(_N, _IN_DIM)
            u, s1, q1 = _mm_stats_in(scale, h, agg, p["W0"], _vec(p["b0"]))
        else:
            agg = _seg_sum(_HID)(
                h, csrc, cldst, cnts).reshape(_N, _HID)
            u0, s0, q0 = _mm_stats_hid(scale, h, agg, p["W1"], _vec(p["b1"]))
            u, s1, q1 = _bnrelu_mm(u0, s0, q0, _vec(p["g_mid"]),
                                   _vec(p["b_mid"]), p["W2"], _vec(p["b2"]))
        z, s2, q2 = _bnrelu_scale(u, s1, q1, _vec(p["g_app"]),
                                  _vec(p["b_app"]), snorm_n)
        if i == 0:
            h = _bnrelu(z, s2, q2, _vec(p["g_node"]), _vec(p["b_node"]))
        else:
            h = _bnrelu_res(z, s2, q2, _vec(p["g_node"]), _vec(p["b_node"]),
                            h_in)
    return h
